# baseline (speedup 1.0000x reference)
import functools

import jax
import jax.numpy as jnp
from jax import lax
from jax.experimental import pallas as pl
from jax.experimental.pallas import tpu as pltpu
from jax.experimental.pallas import tpu_sc as plsc

N = 50000
E = 800000
VOCAB = 10000
D = 64
H1 = 128
H2 = 64
G = 512

HALF = 32
NC = 2
NS = 16
NW = NC * NS

NP = 50176
NSG = 49
EP = 802816
ESG = 784
GP = 528
NB = NP // 512

_MESH = plsc.VectorSubcoreMesh(
    core_axis_name="c", subcore_axis_name="s", num_cores=NC, num_subcores=NS)
_SC_PARAMS = pltpu.CompilerParams(needs_layout_passes=False,
                                  use_tc_tiling_on_sc=False)


_CE = 3136
_CB = 1568


@functools.partial(
    pl.kernel, mesh=_MESH,
    out_type=(jax.ShapeDtypeStruct((NW * NP,), jnp.float32),
              jax.ShapeDtypeStruct((NW * GP,), jnp.float32),
              jax.ShapeDtypeStruct((NC, NP, HALF), jnp.float32)),
    scratch_types=[pltpu.VMEM((NP,), jnp.float32),
                   pltpu.VMEM((_CE,), jnp.int32),
                   pltpu.VMEM((8, 128), jnp.int32),
                   pltpu.VMEM((8 * 128, HALF), jnp.float32),
                   pltpu.SemaphoreType.DMA],
    compiler_params=_SC_PARAMS,
)
def _prep(dst_hbm, batch_hbm, tab_hbm, x2_hbm,
          deg_hbm, cnt_hbm, h0_hbm, histv, idxv, gidx, rows, sem):
    c = lax.axis_index("c")
    tabc_hbm = tab_hbm.at[c]
    s = lax.axis_index("s")
    w = s * NC + c
    ones = jnp.ones((16,), jnp.float32)
    zeros = jnp.zeros((16,), jnp.float32)

    def hist(idx_hbm, nbins, ce, n_outer, base, out_hbm, obase):
        def zero_body(i, _):
            histv[pl.ds(i * 16, 16)] = zeros
            return 0
        lax.fori_loop(0, nbins // 16, zero_body, 0)

        def outer(o, _):
            pltpu.sync_copy(idx_hbm.at[pl.ds(base + o * ce, ce)],
                            idxv.at[pl.ds(0, ce)])

            def inner(k, _):
                v = idxv[pl.ds(k * 16, 16)]
                plsc.addupdate_scatter(histv, [v], ones)
                return 0
            lax.fori_loop(0, ce // 16, inner, 0)
            return 0
        lax.fori_loop(0, n_outer, outer, 0)
        pltpu.sync_copy(histv.at[pl.ds(0, nbins)],
                        out_hbm.at[pl.ds(obase, nbins)])

    hist(dst_hbm, NP, _CE, 8, w * (EP // NW), deg_hbm, w * NP)
    hist(batch_hbm, GP, _CB, 1, w * _CB, cnt_hbm, w * GP)

    for r in range(4):
        sg = s + r * NS

        @pl.when(sg < NSG)
        def _():
            pltpu.sync_copy(x2_hbm.at[sg], gidx)
            descs = []
            for j in range(8):
                descs.append(pltpu.async_copy(
                    tabc_hbm.at[gidx.at[j]],
                    rows.at[pl.ds(j * 128, 128)], sem))
            for d in descs:
                d.wait()
            pltpu.sync_copy(rows, h0_hbm.at[c, pl.ds(sg * 1024, 1024)])


@functools.partial(
    pl.kernel, mesh=_MESH,
    out_type=jax.ShapeDtypeStruct((NC, NP, HALF), jnp.float32),
    scratch_types=[pltpu.VMEM_SHARED((NP, HALF), jnp.float32),
                   pltpu.VMEM((32, 128), jnp.int32),
                   pltpu.VMEM((768, HALF), jnp.float32),
                   pltpu.SemaphoreType.DMA,
                   [pltpu.SemaphoreType.DMA] * 3,
                   [pltpu.SemaphoreType.DMA] * 3],
    compiler_params=_SC_PARAMS,
)
def _spmm(g_hbm, ecomb_hbm, out_hbm, acc, eidx, rows, isem, gsems, ssems):
    c = lax.axis_index("c")
    s = lax.axis_index("s")
    gc_hbm = g_hbm.at[c]

    @pl.when(s < 14)
    def _():
        row0 = s * 3584
        for q in range(7):
            off = row0 + q * 512
            pltpu.sync_copy(gc_hbm.at[pl.ds(off, 512)],
                            rows.at[pl.ds(0, 512)])
            pltpu.sync_copy(rows.at[pl.ds(0, 512)], acc.at[pl.ds(off, 512)])
    plsc.subcore_barrier()

    base_sg = s * 49
    pltpu.async_copy(ecomb_hbm.at[base_sg], eidx.at[pl.ds(0, 16)], isem)

    SLOT = (0, 256, 512, 0)

    def fire_gather(ib, k, sem):
        ds = []
        for t in range(2):
            ds.append(pltpu.async_copy(
                gc_hbm.at[eidx.at[ib + 2 * k + t]],
                rows.at[pl.ds(SLOT[k] + t * 128, 128)], sem))
        return ds

    def fire_scatter(ib, k, sem):
        ds = []
        for t in range(2):
            ds.append(pltpu.async_copy(
                rows.at[pl.ds(SLOT[k] + t * 128, 128)],
                acc.at[eidx.at[ib + 8 + 2 * k + t]], sem, add=True))
        return ds

    def blk(o, _):
        ip = (o % 2) * 16
        pltpu.make_async_copy(ecomb_hbm.at[base_sg],
                              eidx.at[pl.ds(ip, 16)], isem).wait()

        @pl.when(o < 48)
        def _():
            pltpu.async_copy(ecomb_hbm.at[base_sg + o + 1],
                             eidx.at[pl.ds(16 - ip, 16)], isem)

        g0 = fire_gather(ip, 0, gsems[0])
        g1 = fire_gather(ip, 1, gsems[1])
        for d in g0:
            d.wait()
        s0 = fire_scatter(ip, 0, ssems[0])
        g2 = fire_gather(ip, 2, gsems[2])
        for d in g1:
            d.wait()
        s1 = fire_scatter(ip, 1, ssems[1])
        for d in s0:
            d.wait()
        g3 = fire_gather(ip, 3, gsems[0])
        for d in g2:
            d.wait()
        s2 = fire_scatter(ip, 2, ssems[2])
        for d in g3:
            d.wait()
        s3 = fire_scatter(ip, 3, ssems[0])
        for d in s1 + s2 + s3:
            d.wait()
        return 0
    lax.fori_loop(0, 49, blk, 0)

    plsc.subcore_barrier()

    @pl.when(s < 14)
    def _():
        row0 = s * 3584
        for q in range(7):
            off = row0 + q * 512
            pltpu.sync_copy(acc.at[pl.ds(off, 512)], rows.at[pl.ds(0, 512)])
            pltpu.sync_copy(rows.at[pl.ds(0, 512)],
                            out_hbm.at[c, pl.ds(off, 512)])


@functools.partial(
    pl.kernel, mesh=_MESH,
    out_type=jax.ShapeDtypeStruct((NC, GP, HALF), jnp.float32),
    scratch_types=[pltpu.VMEM_SHARED((NP, HALF), jnp.float32),
                   pltpu.VMEM_SHARED((GP, HALF), jnp.float32),
                   pltpu.VMEM((32, 128), jnp.int32),
                   pltpu.VMEM((768, HALF), jnp.float32),
                   pltpu.VMEM((64,), jnp.float32),
                   pltpu.SemaphoreType.DMA,
                   [pltpu.SemaphoreType.DMA] * 3,
                   [pltpu.SemaphoreType.DMA] * 3],
    compiler_params=_SC_PARAMS,
)
def _spmm_pool(g_hbm, ecomb_hbm, batch_hbm, dinvb_hbm, b2s_hbm,
               zeros_hbm, out_hbm,
               acc, pacc, eidx, rows, b2v, isem, gsems, ssems):
    c = lax.axis_index("c")
    s = lax.axis_index("s")
    gc_hbm = g_hbm.at[c]

    pltpu.sync_copy(b2s_hbm, b2v)

    @pl.when(s == 15)
    def _():
        pltpu.sync_copy(zeros_hbm, rows.at[pl.ds(0, GP)])
        pltpu.sync_copy(rows.at[pl.ds(0, GP)], pacc)

    @pl.when(s < 14)
    def _():
        row0 = s * 3584
        for q in range(7):
            off = row0 + q * 512
            pltpu.sync_copy(gc_hbm.at[pl.ds(off, 512)],
                            rows.at[pl.ds(0, 512)])
            pltpu.sync_copy(rows.at[pl.ds(0, 512)], acc.at[pl.ds(off, 512)])
    plsc.subcore_barrier()

    base_sg = s * 49
    pltpu.async_copy(ecomb_hbm.at[base_sg], eidx.at[pl.ds(0, 16)], isem)
    SLOT = (0, 256, 512, 0)

    def fire_gather(ib, k, sem):
        ds = []
        for t in range(2):
            ds.append(pltpu.async_copy(
                gc_hbm.at[eidx.at[ib + 2 * k + t]],
                rows.at[pl.ds(SLOT[k] + t * 128, 128)], sem))
        return ds

    def fire_scatter(ib, k, sem):
        ds = []
        for t in range(2):
            ds.append(pltpu.async_copy(
                rows.at[pl.ds(SLOT[k] + t * 128, 128)],
                acc.at[eidx.at[ib + 8 + 2 * k + t]], sem, add=True))
        return ds

    def blk(o, _):
        ip = (o % 2) * 16
        pltpu.make_async_copy(ecomb_hbm.at[base_sg],
                              eidx.at[pl.ds(ip, 16)], isem).wait()

        @pl.when(o < 48)
        def _():
            pltpu.async_copy(ecomb_hbm.at[base_sg + o + 1],
                             eidx.at[pl.ds(16 - ip, 16)], isem)

        g0 = fire_gather(ip, 0, gsems[0])
        g1 = fire_gather(ip, 1, gsems[1])
        for d in g0:
            d.wait()
        s0 = fire_scatter(ip, 0, ssems[0])
        g2 = fire_gather(ip, 2, gsems[2])
        for d in g1:
            d.wait()
        s1 = fire_scatter(ip, 1, ssems[1])
        for d in s0:
            d.wait()
        g3 = fire_gather(ip, 3, gsems[0])
        for d in g2:
            d.wait()
        s2 = fire_scatter(ip, 2, ssems[2])
        for d in g3:
            d.wait()
        s3 = fire_scatter(ip, 3, ssems[0])
        for d in s1 + s2 + s3:
            d.wait()
        return 0
    lax.fori_loop(0, 49, blk, 0)

    plsc.subcore_barrier()

    vb0 = b2v[pl.ds(c * HALF, 16)]
    vb1 = b2v[pl.ds(c * HALF + 16, 16)]
    for r in range(4):
        sg = s + r * NS

        @pl.when(sg < NSG)
        def _():
            pltpu.sync_copy(batch_hbm.at[sg], eidx.at[pl.ds(0, 8)])
            for j in range(8):
                row0 = sg * 1024 + j * 128
                pltpu.sync_copy(acc.at[pl.ds(row0, 128)],
                                rows.at[pl.ds(0, 128)])
                pltpu.sync_copy(dinvb_hbm.at[pl.ds(row0, 128)],
                                rows.at[pl.ds(128, 128)])

                def ew(i, _):
                    a0 = rows[i, pl.ds(0, 16)] * rows[128 + i, pl.ds(0, 16)]
                    a1 = rows[i, pl.ds(16, 16)] * rows[128 + i, pl.ds(16, 16)]
                    rows[i, pl.ds(0, 16)] = jnp.maximum(a0 + vb0, 0.0)
                    rows[i, pl.ds(16, 16)] = jnp.maximum(a1 + vb1, 0.0)
                    return 0
                lax.fori_loop(0, 128, ew, 0)
                pltpu.sync_copy(rows.at[pl.ds(0, 128)],
                                pacc.at[eidx.at[j]], add=True)

    plsc.subcore_barrier()

    @pl.when(s == 15)
    def _():
        pltpu.sync_copy(pacc, rows.at[pl.ds(0, GP)])
        pltpu.sync_copy(rows.at[pl.ds(0, GP)], out_hbm.at[c])


def _tc_call(body, grid, in_specs, out_specs, out_shape):
    return pl.pallas_call(body, grid=grid, in_specs=in_specs,
                          out_specs=out_specs, out_shape=out_shape)


_ONES_OUTER = (((0,), (0,)), ((), ()))


def _scale_body(p_ref, h_ref, g_ref, d_ref):
    s = jnp.sum(p_ref[...], axis=0, keepdims=True)
    dinv = lax.rsqrt(1.0 + s)
    db = lax.dot_general(dinv, jnp.ones((1, HALF), jnp.float32),
                         _ONES_OUTER,
                         preferred_element_type=jnp.float32)
    d_ref[...] = db
    g_ref[...] = h_ref[...] * db[None]


def _mlp_body(s1_ref, d_ref, w1_ref, b1_ref, w2a_ref, w2b_ref, o_ref):
    d = d_ref[...]
    p = s1_ref[...] * d[None]
    h1 = jnp.maximum(
        jnp.dot(p[0], w1_ref[0], preferred_element_type=jnp.float32)
        + jnp.dot(p[1], w1_ref[1], preferred_element_type=jnp.float32)
        + b1_ref[...], 0.0)
    o_ref[0, ...] = jnp.dot(
        h1, w2a_ref[...], preferred_element_type=jnp.float32) * d
    o_ref[1, ...] = jnp.dot(
        h1, w2b_ref[...], preferred_element_type=jnp.float32) * d


def _mean_body(p_ref, c_ref, o_ref):
    cnt = jnp.sum(c_ref[...], axis=0, keepdims=True)[:, :G]
    ic = lax.dot_general(1.0 / jnp.maximum(cnt, 1.0),
                         jnp.ones((1, HALF), jnp.float32), _ONES_OUTER,
                         preferred_element_type=jnp.float32)
    o_ref[...] = jnp.concatenate([p_ref[0] * ic, p_ref[1] * ic], axis=1)


def kernel(x, edge_index, batch, emb, W1, b1, W2, b2):
    f32 = jnp.float32
    i32 = jnp.int32

    x0 = x[:, 0]
    xp = jnp.concatenate([x0, jnp.zeros((NP - N,), i32)])
    x2 = xp.reshape(NSG, 8, 128)

    src = edge_index[0]
    dst = edge_index[1]
    srcp = jnp.concatenate([src, jnp.zeros((EP - E,), i32)])
    dstp = jnp.concatenate([dst, jnp.full((EP - E,), N, i32)])
    ecomb = jnp.concatenate([srcp.reshape(ESG, 8, 128),
                             dstp.reshape(ESG, 8, 128)],
                            axis=1)

    batchp = jnp.concatenate([batch, jnp.full((NP - N,), G, i32)])
    batch2 = batchp.reshape(NSG, 8, 128)

    tab = emb.reshape(VOCAB, NC, HALF).transpose(1, 0, 2)

    deg_parts, cnt_parts, h0 = _prep(dstp, batchp, tab, x2)
    deg_parts = deg_parts.reshape(NW, NP)
    cnt_parts = cnt_parts.reshape(NW, GP)

    espec = pl.BlockSpec((NC, 512, HALF), lambda i: (0, i, 0))
    dspec = pl.BlockSpec((512, HALF), lambda i: (i, 0))
    eshape = jax.ShapeDtypeStruct((NC, NP, HALF), f32)

    g1, dinvb = _tc_call(
        _scale_body, (NB,),
        [pl.BlockSpec((NW, 512), lambda i: (0, i)), espec],
        [espec, dspec],
        [eshape, jax.ShapeDtypeStruct((NP, HALF), f32)],
    )(deg_parts, h0)

    s1 = _spmm(g1, ecomb)

    g2 = _tc_call(
        _mlp_body, (NB,),
        [espec, dspec,
         pl.BlockSpec((NC, HALF, H1), lambda i: (0, 0, 0)),
         pl.BlockSpec((1, H1), lambda i: (0, 0)),
         pl.BlockSpec((H1, HALF), lambda i: (0, 0)),
         pl.BlockSpec((H1, HALF), lambda i: (0, 0))],
        espec, eshape,
    )(s1, dinvb, W1.reshape(NC, HALF, H1), b1.reshape(1, H1),
      W2[:, :HALF], W2[:, HALF:])

    pooled = _spmm_pool(g2, ecomb, batch2,
                        dinvb, b2, jnp.zeros((GP, HALF), f32))

    out = _tc_call(
        _mean_body, (1,),
        [pl.BlockSpec((NC, G, HALF), lambda i: (0, 0, 0)),
         pl.BlockSpec((NW, GP), lambda i: (0, 0))],
        pl.BlockSpec((G, H2), lambda i: (0, 0)),
        jax.ShapeDtypeStruct((G, H2), f32),
    )(pooled[:, :G, :], cnt_parts)

    return out


# speedup vs baseline: 1.0505x; 1.0505x over previous
import functools

import jax
import jax.numpy as jnp
from jax import lax
from jax.experimental import pallas as pl
from jax.experimental.pallas import tpu as pltpu
from jax.experimental.pallas import tpu_sc as plsc

N = 50000
E = 800000
VOCAB = 10000
D = 64
H1 = 128
H2 = 64
G = 512

HALF = 32
NC = 2
NS = 16
NW = NC * NS

NP = 50176
NSG = 49
EP = 802816
ESG = 784
GP = 528
NB = NP // 1024

_MESH = plsc.VectorSubcoreMesh(
    core_axis_name="c", subcore_axis_name="s", num_cores=NC, num_subcores=NS)
_SC_PARAMS = pltpu.CompilerParams(needs_layout_passes=False,
                                  use_tc_tiling_on_sc=False)


_CE = 3136
_CB = 1568


@functools.partial(
    pl.kernel, mesh=_MESH,
    out_type=(jax.ShapeDtypeStruct((NW * NP,), jnp.float32),
              jax.ShapeDtypeStruct((NW * GP,), jnp.float32),
              jax.ShapeDtypeStruct((NC, NP, HALF), jnp.float32)),
    scratch_types=[pltpu.VMEM((NP,), jnp.float32),
                   pltpu.VMEM((_CE,), jnp.int32),
                   pltpu.VMEM((8, 128), jnp.int32),
                   pltpu.VMEM((8 * 128, HALF), jnp.float32),
                   pltpu.SemaphoreType.DMA],
    compiler_params=_SC_PARAMS,
)
def _prep(dst_hbm, batch_hbm, tab_hbm, x2_hbm,
          deg_hbm, cnt_hbm, h0_hbm, histv, idxv, gidx, rows, sem):
    c = lax.axis_index("c")
    tabc_hbm = tab_hbm.at[c]
    s = lax.axis_index("s")
    w = s * NC + c
    ones = jnp.ones((16,), jnp.float32)
    zeros = jnp.zeros((16,), jnp.float32)

    def hist(idx_hbm, nbins, ce, n_outer, base, out_hbm, obase):
        def zero_body(i, _):
            histv[pl.ds(i * 16, 16)] = zeros
            return 0
        lax.fori_loop(0, nbins // 16, zero_body, 0)

        def outer(o, _):
            pltpu.sync_copy(idx_hbm.at[pl.ds(base + o * ce, ce)],
                            idxv.at[pl.ds(0, ce)])

            def inner(k, _):
                v = idxv[pl.ds(k * 16, 16)]
                plsc.addupdate_scatter(histv, [v], ones)
                return 0
            lax.fori_loop(0, ce // 16, inner, 0)
            return 0
        lax.fori_loop(0, n_outer, outer, 0)
        pltpu.sync_copy(histv.at[pl.ds(0, nbins)],
                        out_hbm.at[pl.ds(obase, nbins)])

    hist(dst_hbm, NP, _CE, 8, w * (EP // NW), deg_hbm, w * NP)
    hist(batch_hbm, GP, _CB, 1, w * _CB, cnt_hbm, w * GP)

    for r in range(4):
        sg = s + r * NS

        @pl.when(sg < NSG)
        def _():
            pltpu.sync_copy(x2_hbm.at[sg], gidx)
            descs = []
            for j in range(8):
                descs.append(pltpu.async_copy(
                    tabc_hbm.at[gidx.at[j]],
                    rows.at[pl.ds(j * 128, 128)], sem))
            for d in descs:
                d.wait()
            pltpu.sync_copy(rows, h0_hbm.at[c, pl.ds(sg * 1024, 1024)])


@functools.partial(
    pl.kernel, mesh=_MESH,
    out_type=jax.ShapeDtypeStruct((NC, NP, HALF), jnp.float32),
    scratch_types=[pltpu.VMEM_SHARED((NP, HALF), jnp.float32),
                   pltpu.VMEM((32, 128), jnp.int32),
                   pltpu.VMEM((768, HALF), jnp.float32),
                   pltpu.SemaphoreType.DMA,
                   [pltpu.SemaphoreType.DMA] * 3,
                   [pltpu.SemaphoreType.DMA] * 3],
    compiler_params=_SC_PARAMS,
)
def _spmm(g_hbm, ecomb_hbm, out_hbm, acc, eidx, rows, isem, gsems, ssems):
    c = lax.axis_index("c")
    s = lax.axis_index("s")
    gc_hbm = g_hbm.at[c]

    @pl.when(s < 14)
    def _():
        row0 = s * 3584
        for q in range(7):
            off = row0 + q * 512
            pltpu.sync_copy(gc_hbm.at[pl.ds(off, 512)],
                            rows.at[pl.ds(0, 512)])
            pltpu.sync_copy(rows.at[pl.ds(0, 512)], acc.at[pl.ds(off, 512)])
    plsc.subcore_barrier()

    base_sg = s * 49
    pltpu.async_copy(ecomb_hbm.at[base_sg], eidx.at[pl.ds(0, 16)], isem)

    SLOT = (0, 256, 512, 0)

    def fire_gather(ib, k, sem):
        ds = []
        for t in range(2):
            ds.append(pltpu.async_copy(
                gc_hbm.at[eidx.at[ib + 2 * k + t]],
                rows.at[pl.ds(SLOT[k] + t * 128, 128)], sem))
        return ds

    def fire_scatter(ib, k, sem):
        ds = []
        for t in range(2):
            ds.append(pltpu.async_copy(
                rows.at[pl.ds(SLOT[k] + t * 128, 128)],
                acc.at[eidx.at[ib + 8 + 2 * k + t]], sem, add=True))
        return ds

    def blk(o, _):
        ip = (o % 2) * 16
        pltpu.make_async_copy(ecomb_hbm.at[base_sg],
                              eidx.at[pl.ds(ip, 16)], isem).wait()

        @pl.when(o < 48)
        def _():
            pltpu.async_copy(ecomb_hbm.at[base_sg + o + 1],
                             eidx.at[pl.ds(16 - ip, 16)], isem)

        g0 = fire_gather(ip, 0, gsems[0])
        g1 = fire_gather(ip, 1, gsems[1])
        for d in g0:
            d.wait()
        s0 = fire_scatter(ip, 0, ssems[0])
        g2 = fire_gather(ip, 2, gsems[2])
        for d in g1:
            d.wait()
        s1 = fire_scatter(ip, 1, ssems[1])
        for d in s0:
            d.wait()
        g3 = fire_gather(ip, 3, gsems[0])
        for d in g2:
            d.wait()
        s2 = fire_scatter(ip, 2, ssems[2])
        for d in g3:
            d.wait()
        s3 = fire_scatter(ip, 3, ssems[0])
        for d in s1 + s2 + s3:
            d.wait()
        return 0
    lax.fori_loop(0, 49, blk, 0)

    plsc.subcore_barrier()

    @pl.when(s < 14)
    def _():
        row0 = s * 3584
        for q in range(7):
            off = row0 + q * 512
            pltpu.sync_copy(acc.at[pl.ds(off, 512)], rows.at[pl.ds(0, 512)])
            pltpu.sync_copy(rows.at[pl.ds(0, 512)],
                            out_hbm.at[c, pl.ds(off, 512)])


@functools.partial(
    pl.kernel, mesh=_MESH,
    out_type=jax.ShapeDtypeStruct((NC, GP, HALF), jnp.float32),
    scratch_types=[pltpu.VMEM_SHARED((NP, HALF), jnp.float32),
                   pltpu.VMEM_SHARED((GP, HALF), jnp.float32),
                   pltpu.VMEM((32, 128), jnp.int32),
                   pltpu.VMEM((768, HALF), jnp.float32),
                   pltpu.VMEM((64,), jnp.float32),
                   pltpu.SemaphoreType.DMA,
                   [pltpu.SemaphoreType.DMA] * 3,
                   [pltpu.SemaphoreType.DMA] * 3],
    compiler_params=_SC_PARAMS,
)
def _spmm_pool(g_hbm, ecomb_hbm, batch_hbm, dinvb_hbm, b2s_hbm,
               zeros_hbm, out_hbm,
               acc, pacc, eidx, rows, b2v, isem, gsems, ssems):
    c = lax.axis_index("c")
    s = lax.axis_index("s")
    gc_hbm = g_hbm.at[c]

    pltpu.sync_copy(b2s_hbm, b2v)

    @pl.when(s == 15)
    def _():
        pltpu.sync_copy(zeros_hbm, rows.at[pl.ds(0, GP)])
        pltpu.sync_copy(rows.at[pl.ds(0, GP)], pacc)

    @pl.when(s < 14)
    def _():
        row0 = s * 3584
        for q in range(7):
            off = row0 + q * 512
            pltpu.sync_copy(gc_hbm.at[pl.ds(off, 512)],
                            rows.at[pl.ds(0, 512)])
            pltpu.sync_copy(rows.at[pl.ds(0, 512)], acc.at[pl.ds(off, 512)])
    plsc.subcore_barrier()

    base_sg = s * 49
    pltpu.async_copy(ecomb_hbm.at[base_sg], eidx.at[pl.ds(0, 16)], isem)
    SLOT = (0, 256, 512, 0)

    def fire_gather(ib, k, sem):
        ds = []
        for t in range(2):
            ds.append(pltpu.async_copy(
                gc_hbm.at[eidx.at[ib + 2 * k + t]],
                rows.at[pl.ds(SLOT[k] + t * 128, 128)], sem))
        return ds

    def fire_scatter(ib, k, sem):
        ds = []
        for t in range(2):
            ds.append(pltpu.async_copy(
                rows.at[pl.ds(SLOT[k] + t * 128, 128)],
                acc.at[eidx.at[ib + 8 + 2 * k + t]], sem, add=True))
        return ds

    def blk(o, _):
        ip = (o % 2) * 16
        pltpu.make_async_copy(ecomb_hbm.at[base_sg],
                              eidx.at[pl.ds(ip, 16)], isem).wait()

        @pl.when(o < 48)
        def _():
            pltpu.async_copy(ecomb_hbm.at[base_sg + o + 1],
                             eidx.at[pl.ds(16 - ip, 16)], isem)

        g0 = fire_gather(ip, 0, gsems[0])
        g1 = fire_gather(ip, 1, gsems[1])
        for d in g0:
            d.wait()
        s0 = fire_scatter(ip, 0, ssems[0])
        g2 = fire_gather(ip, 2, gsems[2])
        for d in g1:
            d.wait()
        s1 = fire_scatter(ip, 1, ssems[1])
        for d in s0:
            d.wait()
        g3 = fire_gather(ip, 3, gsems[0])
        for d in g2:
            d.wait()
        s2 = fire_scatter(ip, 2, ssems[2])
        for d in g3:
            d.wait()
        s3 = fire_scatter(ip, 3, ssems[0])
        for d in s1 + s2 + s3:
            d.wait()
        return 0
    lax.fori_loop(0, 49, blk, 0)

    plsc.subcore_barrier()

    vb0 = b2v[pl.ds(c * HALF, 16)]
    vb1 = b2v[pl.ds(c * HALF + 16, 16)]
    for r in range(4):
        sg = s + r * NS

        @pl.when(sg < NSG)
        def _():
            pltpu.sync_copy(batch_hbm.at[sg], eidx.at[pl.ds(0, 8)])
            for j in range(8):
                row0 = sg * 1024 + j * 128
                pltpu.sync_copy(acc.at[pl.ds(row0, 128)],
                                rows.at[pl.ds(0, 128)])
                pltpu.sync_copy(dinvb_hbm.at[pl.ds(row0, 128)],
                                rows.at[pl.ds(128, 128)])

                def ew(i, _):
                    a0 = rows[i, pl.ds(0, 16)] * rows[128 + i, pl.ds(0, 16)]
                    a1 = rows[i, pl.ds(16, 16)] * rows[128 + i, pl.ds(16, 16)]
                    rows[i, pl.ds(0, 16)] = jnp.maximum(a0 + vb0, 0.0)
                    rows[i, pl.ds(16, 16)] = jnp.maximum(a1 + vb1, 0.0)
                    return 0
                lax.fori_loop(0, 128, ew, 0)
                pltpu.sync_copy(rows.at[pl.ds(0, 128)],
                                pacc.at[eidx.at[j]], add=True)

    plsc.subcore_barrier()

    @pl.when(s == 15)
    def _():
        pltpu.sync_copy(pacc, rows.at[pl.ds(0, GP)])
        pltpu.sync_copy(rows.at[pl.ds(0, GP)], out_hbm.at[c])


def _tc_call(body, grid, in_specs, out_specs, out_shape):
    return pl.pallas_call(body, grid=grid, in_specs=in_specs,
                          out_specs=out_specs, out_shape=out_shape)


_ONES_OUTER = (((0,), (0,)), ((), ()))


def _scale_body(p_ref, h_ref, g_ref, d_ref):
    s = jnp.sum(p_ref[...], axis=0, keepdims=True)
    dinv = lax.rsqrt(1.0 + s)
    db = lax.dot_general(dinv, jnp.ones((1, HALF), jnp.float32),
                         _ONES_OUTER, precision=lax.Precision.HIGHEST,
                         preferred_element_type=jnp.float32)
    d_ref[...] = db
    g_ref[...] = h_ref[...] * db[None]


def _mlp_body(s1_ref, d_ref, w1_ref, b1_ref, w2a_ref, w2b_ref, o_ref):
    d = d_ref[...]
    p = s1_ref[...] * d[None]
    h1 = jnp.maximum(
        jnp.dot(p[0], w1_ref[0], preferred_element_type=jnp.float32)
        + jnp.dot(p[1], w1_ref[1], preferred_element_type=jnp.float32)
        + b1_ref[...], 0.0)
    o_ref[0, ...] = jnp.dot(
        h1, w2a_ref[...], preferred_element_type=jnp.float32) * d
    o_ref[1, ...] = jnp.dot(
        h1, w2b_ref[...], preferred_element_type=jnp.float32) * d


def _mean_body(p_ref, c_ref, o_ref):
    cnt = jnp.sum(c_ref[...], axis=0, keepdims=True)[:, :G]
    ic = lax.dot_general(1.0 / jnp.maximum(cnt, 1.0),
                         jnp.ones((1, HALF), jnp.float32), _ONES_OUTER,
                         precision=lax.Precision.HIGHEST,
                         preferred_element_type=jnp.float32)
    o_ref[...] = jnp.concatenate([p_ref[0] * ic, p_ref[1] * ic], axis=1)


def kernel(x, edge_index, batch, emb, W1, b1, W2, b2):
    f32 = jnp.float32
    i32 = jnp.int32

    x0 = x[:, 0]
    xp = jnp.concatenate([x0, jnp.zeros((NP - N,), i32)])
    x2 = xp.reshape(NSG, 8, 128)

    src = edge_index[0]
    dst = edge_index[1]
    srcp = jnp.concatenate([src, jnp.zeros((EP - E,), i32)])
    dstp = jnp.concatenate([dst, jnp.full((EP - E,), N, i32)])
    ecomb = jnp.concatenate([srcp.reshape(ESG, 8, 128),
                             dstp.reshape(ESG, 8, 128)],
                            axis=1)

    batchp = jnp.concatenate([batch, jnp.full((NP - N,), G, i32)])
    batch2 = batchp.reshape(NSG, 8, 128)

    tab = emb.reshape(VOCAB, NC, HALF).transpose(1, 0, 2)

    deg_parts, cnt_parts, h0 = _prep(dstp, batchp, tab, x2)
    deg_parts = deg_parts.reshape(NW, NP)
    cnt_parts = cnt_parts.reshape(NW, GP)

    espec = pl.BlockSpec((NC, 1024, HALF), lambda i: (0, i, 0))
    dspec = pl.BlockSpec((1024, HALF), lambda i: (i, 0))
    eshape = jax.ShapeDtypeStruct((NC, NP, HALF), f32)

    g1, dinvb = _tc_call(
        _scale_body, (NB,),
        [pl.BlockSpec((NW, 1024), lambda i: (0, i)), espec],
        [espec, dspec],
        [eshape, jax.ShapeDtypeStruct((NP, HALF), f32)],
    )(deg_parts, h0)

    s1 = _spmm(g1, ecomb)

    g2 = _tc_call(
        _mlp_body, (NB,),
        [espec, dspec,
         pl.BlockSpec((NC, HALF, H1), lambda i: (0, 0, 0)),
         pl.BlockSpec((1, H1), lambda i: (0, 0)),
         pl.BlockSpec((H1, HALF), lambda i: (0, 0)),
         pl.BlockSpec((H1, HALF), lambda i: (0, 0))],
        espec, eshape,
    )(s1, dinvb, W1.reshape(NC, HALF, H1), b1.reshape(1, H1),
      W2[:, :HALF], W2[:, HALF:])

    pooled = _spmm_pool(g2, ecomb, batch2,
                        dinvb, b2, jnp.zeros((GP, HALF), f32))

    out = _tc_call(
        _mean_body, (1,),
        [pl.BlockSpec((NC, G, HALF), lambda i: (0, 0, 0)),
         pl.BlockSpec((NW, GP), lambda i: (0, 0))],
        pl.BlockSpec((G, H2), lambda i: (0, 0)),
        jax.ShapeDtypeStruct((G, H2), f32),
    )(pooled[:, :G, :], cnt_parts)

    return out


# speedup vs baseline: 1.0651x; 1.0139x over previous
import functools

import jax
import jax.numpy as jnp
from jax import lax
from jax.experimental import pallas as pl
from jax.experimental.pallas import tpu as pltpu
from jax.experimental.pallas import tpu_sc as plsc

N = 50000
E = 800000
VOCAB = 10000
D = 64
H1 = 128
H2 = 64
G = 512

HALF = 32
NC = 2
NS = 16
NW = NC * NS

NP = 50176
NSG = 49
EP = 802816
ESG = 784
GP = 528
NB = NP // 1024

_MESH = plsc.VectorSubcoreMesh(
    core_axis_name="c", subcore_axis_name="s", num_cores=NC, num_subcores=NS)
_SC_PARAMS = pltpu.CompilerParams(needs_layout_passes=False,
                                  use_tc_tiling_on_sc=False)


_CE = 3136
_CB = 1568


@functools.partial(
    pl.kernel, mesh=_MESH,
    out_type=(jax.ShapeDtypeStruct((NW * NP,), jnp.float32),
              jax.ShapeDtypeStruct((NW * GP,), jnp.float32),
              jax.ShapeDtypeStruct((NC, NP, HALF), jnp.float32)),
    scratch_types=[pltpu.VMEM((NP,), jnp.float32),
                   pltpu.VMEM((_CE,), jnp.int32),
                   pltpu.VMEM((8, 128), jnp.int32),
                   pltpu.VMEM((8 * 128, HALF), jnp.float32),
                   pltpu.SemaphoreType.DMA],
    compiler_params=_SC_PARAMS,
)
def _prep(dst_hbm, batch_hbm, tab_hbm, x2_hbm,
          deg_hbm, cnt_hbm, h0_hbm, histv, idxv, gidx, rows, sem):
    c = lax.axis_index("c")
    tabc_hbm = tab_hbm.at[c]
    s = lax.axis_index("s")
    w = s * NC + c
    ones = jnp.ones((16,), jnp.float32)
    zeros = jnp.zeros((16,), jnp.float32)

    def hist(idx_hbm, nbins, ce, n_outer, base, out_hbm, obase):
        def zero_body(i, _):
            histv[pl.ds(i * 16, 16)] = zeros
            return 0
        lax.fori_loop(0, nbins // 16, zero_body, 0)

        def outer(o, _):
            pltpu.sync_copy(idx_hbm.at[pl.ds(base + o * ce, ce)],
                            idxv.at[pl.ds(0, ce)])

            def inner(k, _):
                v = idxv[pl.ds(k * 16, 16)]
                plsc.addupdate_scatter(histv, [v], ones)
                return 0
            lax.fori_loop(0, ce // 16, inner, 0)
            return 0
        lax.fori_loop(0, n_outer, outer, 0)
        pltpu.sync_copy(histv.at[pl.ds(0, nbins)],
                        out_hbm.at[pl.ds(obase, nbins)])

    hist(dst_hbm, NP, _CE, 8, w * (EP // NW), deg_hbm, w * NP)
    hist(batch_hbm, GP, _CB, 1, w * _CB, cnt_hbm, w * GP)

    for r in range(4):
        sg = s + r * NS

        @pl.when(sg < NSG)
        def _():
            pltpu.sync_copy(x2_hbm.at[sg], gidx)
            descs = []
            for j in range(8):
                descs.append(pltpu.async_copy(
                    tabc_hbm.at[gidx.at[j]],
                    rows.at[pl.ds(j * 128, 128)], sem))
            for d in descs:
                d.wait()
            pltpu.sync_copy(rows, h0_hbm.at[c, pl.ds(sg * 1024, 1024)])


@functools.partial(
    pl.kernel, mesh=_MESH,
    out_type=jax.ShapeDtypeStruct((NC, NP, HALF), jnp.float32),
    scratch_types=[pltpu.VMEM_SHARED((NP, HALF), jnp.float32),
                   pltpu.VMEM((32, 128), jnp.int32),
                   pltpu.VMEM((768, HALF), jnp.float32),
                   pltpu.SemaphoreType.DMA,
                   [pltpu.SemaphoreType.DMA] * 3,
                   [pltpu.SemaphoreType.DMA] * 3],
    compiler_params=_SC_PARAMS,
)
def _spmm(g_hbm, ecomb_hbm, out_hbm, acc, eidx, rows, isem, gsems, ssems):
    c = lax.axis_index("c")
    s = lax.axis_index("s")
    gc_hbm = g_hbm.at[c]

    @pl.when(s < 14)
    def _():
        row0 = s * 3584
        for q in range(7):
            off = row0 + q * 512
            pltpu.sync_copy(gc_hbm.at[pl.ds(off, 512)],
                            rows.at[pl.ds(0, 512)])
            pltpu.sync_copy(rows.at[pl.ds(0, 512)], acc.at[pl.ds(off, 512)])
    plsc.subcore_barrier()

    base_sg = s * 49
    pltpu.async_copy(ecomb_hbm.at[base_sg], eidx.at[pl.ds(0, 16)], isem)

    SLOT = (0, 256, 512, 0)

    def fire_gather(ib, k, sem):
        ds = []
        for t in range(2):
            ds.append(pltpu.async_copy(
                gc_hbm.at[eidx.at[ib + 2 * k + t]],
                rows.at[pl.ds(SLOT[k] + t * 128, 128)], sem))
        return ds

    def fire_scatter(ib, k, sem):
        ds = []
        for t in range(2):
            ds.append(pltpu.async_copy(
                rows.at[pl.ds(SLOT[k] + t * 128, 128)],
                acc.at[eidx.at[ib + 8 + 2 * k + t]], sem, add=True))
        return ds

    def blk(o, _):
        ip = (o % 2) * 16
        pltpu.make_async_copy(ecomb_hbm.at[base_sg],
                              eidx.at[pl.ds(ip, 16)], isem).wait()

        @pl.when(o < 48)
        def _():
            pltpu.async_copy(ecomb_hbm.at[base_sg + o + 1],
                             eidx.at[pl.ds(16 - ip, 16)], isem)

        g0 = fire_gather(ip, 0, gsems[0])
        g1 = fire_gather(ip, 1, gsems[1])
        for d in g0:
            d.wait()
        s0 = fire_scatter(ip, 0, ssems[0])
        g2 = fire_gather(ip, 2, gsems[2])
        for d in g1:
            d.wait()
        s1 = fire_scatter(ip, 1, ssems[1])
        for d in s0:
            d.wait()
        g3 = fire_gather(ip, 3, gsems[0])
        for d in g2:
            d.wait()
        s2 = fire_scatter(ip, 2, ssems[2])
        for d in g3:
            d.wait()
        s3 = fire_scatter(ip, 3, ssems[0])
        for d in s1 + s2 + s3:
            d.wait()
        return 0
    lax.fori_loop(0, 49, blk, 0)

    plsc.subcore_barrier()

    @pl.when(s < 14)
    def _():
        row0 = s * 3584
        for q in range(7):
            off = row0 + q * 512
            pltpu.sync_copy(acc.at[pl.ds(off, 512)], rows.at[pl.ds(0, 512)])
            pltpu.sync_copy(rows.at[pl.ds(0, 512)],
                            out_hbm.at[c, pl.ds(off, 512)])


@functools.partial(
    pl.kernel, mesh=_MESH,
    out_type=jax.ShapeDtypeStruct((NC, GP, HALF), jnp.float32),
    scratch_types=[pltpu.VMEM_SHARED((NP, HALF), jnp.float32),
                   pltpu.VMEM_SHARED((GP, HALF), jnp.float32),
                   pltpu.VMEM((32, 128), jnp.int32),
                   pltpu.VMEM((768, HALF), jnp.float32),
                   pltpu.VMEM((64,), jnp.float32),
                   pltpu.SemaphoreType.DMA,
                   [pltpu.SemaphoreType.DMA] * 3,
                   [pltpu.SemaphoreType.DMA] * 3],
    compiler_params=_SC_PARAMS,
)
def _spmm_pool(g_hbm, ecomb_hbm, batch_hbm, dinvb_hbm, b2s_hbm,
               zeros_hbm, out_hbm,
               acc, pacc, eidx, rows, b2v, isem, gsems, ssems):
    c = lax.axis_index("c")
    s = lax.axis_index("s")
    gc_hbm = g_hbm.at[c]

    pltpu.sync_copy(b2s_hbm, b2v)

    @pl.when(s == 15)
    def _():
        pltpu.sync_copy(zeros_hbm, rows.at[pl.ds(0, GP)])
        pltpu.sync_copy(rows.at[pl.ds(0, GP)], pacc)

    @pl.when(s < 14)
    def _():
        row0 = s * 3584
        for q in range(7):
            off = row0 + q * 512
            pltpu.sync_copy(gc_hbm.at[pl.ds(off, 512)],
                            rows.at[pl.ds(0, 512)])
            pltpu.sync_copy(rows.at[pl.ds(0, 512)], acc.at[pl.ds(off, 512)])
    plsc.subcore_barrier()

    base_sg = s * 49
    pltpu.async_copy(ecomb_hbm.at[base_sg], eidx.at[pl.ds(0, 16)], isem)
    SLOT = (0, 256, 512, 0)

    def fire_gather(ib, k, sem):
        ds = []
        for t in range(2):
            ds.append(pltpu.async_copy(
                gc_hbm.at[eidx.at[ib + 2 * k + t]],
                rows.at[pl.ds(SLOT[k] + t * 128, 128)], sem))
        return ds

    def fire_scatter(ib, k, sem):
        ds = []
        for t in range(2):
            ds.append(pltpu.async_copy(
                rows.at[pl.ds(SLOT[k] + t * 128, 128)],
                acc.at[eidx.at[ib + 8 + 2 * k + t]], sem, add=True))
        return ds

    def blk(o, _):
        ip = (o % 2) * 16
        pltpu.make_async_copy(ecomb_hbm.at[base_sg],
                              eidx.at[pl.ds(ip, 16)], isem).wait()

        @pl.when(o < 48)
        def _():
            pltpu.async_copy(ecomb_hbm.at[base_sg + o + 1],
                             eidx.at[pl.ds(16 - ip, 16)], isem)

        g0 = fire_gather(ip, 0, gsems[0])
        g1 = fire_gather(ip, 1, gsems[1])
        for d in g0:
            d.wait()
        s0 = fire_scatter(ip, 0, ssems[0])
        g2 = fire_gather(ip, 2, gsems[2])
        for d in g1:
            d.wait()
        s1 = fire_scatter(ip, 1, ssems[1])
        for d in s0:
            d.wait()
        g3 = fire_gather(ip, 3, gsems[0])
        for d in g2:
            d.wait()
        s2 = fire_scatter(ip, 2, ssems[2])
        for d in g3:
            d.wait()
        s3 = fire_scatter(ip, 3, ssems[0])
        for d in s1 + s2 + s3:
            d.wait()
        return 0
    lax.fori_loop(0, 49, blk, 0)

    plsc.subcore_barrier()

    vb0 = b2v[pl.ds(c * HALF, 16)]
    vb1 = b2v[pl.ds(c * HALF + 16, 16)]
    for r in range(4):
        sg = s + r * NS

        @pl.when(sg < NSG)
        def _():
            pltpu.sync_copy(batch_hbm.at[sg], eidx.at[pl.ds(0, 8)])
            for q in range(4):
                row0 = sg * 1024 + q * 256
                pltpu.sync_copy(acc.at[pl.ds(row0, 256)],
                                rows.at[pl.ds(0, 256)])
                pltpu.sync_copy(dinvb_hbm.at[pl.ds(row0, 256)],
                                rows.at[pl.ds(256, 256)])

                def ew(i, _):
                    a0 = rows[i, pl.ds(0, 16)] * rows[256 + i, pl.ds(0, 16)]
                    a1 = rows[i, pl.ds(16, 16)] * rows[256 + i, pl.ds(16, 16)]
                    rows[i, pl.ds(0, 16)] = jnp.maximum(a0 + vb0, 0.0)
                    rows[i, pl.ds(16, 16)] = jnp.maximum(a1 + vb1, 0.0)
                    return 0
                lax.fori_loop(0, 256, ew, 0)
                for t in range(2):
                    pltpu.sync_copy(
                        rows.at[pl.ds(t * 128, 128)],
                        pacc.at[eidx.at[q * 2 + t]], add=True)

    plsc.subcore_barrier()

    @pl.when(s == 15)
    def _():
        pltpu.sync_copy(pacc, rows.at[pl.ds(0, GP)])
        pltpu.sync_copy(rows.at[pl.ds(0, GP)], out_hbm.at[c])


def _tc_call(body, grid, in_specs, out_specs, out_shape):
    return pl.pallas_call(body, grid=grid, in_specs=in_specs,
                          out_specs=out_specs, out_shape=out_shape)


_ONES_OUTER = (((0,), (0,)), ((), ()))


def _scale_body(p_ref, h_ref, g_ref, d_ref):
    s = jnp.sum(p_ref[...], axis=0, keepdims=True)
    dinv = lax.rsqrt(1.0 + s)
    db = lax.dot_general(dinv, jnp.ones((1, HALF), jnp.float32),
                         _ONES_OUTER, precision=lax.Precision.HIGHEST,
                         preferred_element_type=jnp.float32)
    d_ref[...] = db
    g_ref[...] = h_ref[...] * db[None]


def _mlp_body(s1_ref, d_ref, w1_ref, b1_ref, w2a_ref, w2b_ref, o_ref):
    d = d_ref[...]
    p = s1_ref[...] * d[None]
    h1 = jnp.maximum(
        jnp.dot(p[0], w1_ref[0], preferred_element_type=jnp.float32)
        + jnp.dot(p[1], w1_ref[1], preferred_element_type=jnp.float32)
        + b1_ref[...], 0.0)
    o_ref[0, ...] = jnp.dot(
        h1, w2a_ref[...], preferred_element_type=jnp.float32) * d
    o_ref[1, ...] = jnp.dot(
        h1, w2b_ref[...], preferred_element_type=jnp.float32) * d


def _mean_body(p_ref, c_ref, o_ref):
    cnt = jnp.sum(c_ref[...], axis=0, keepdims=True)[:, :G]
    ic = lax.dot_general(1.0 / jnp.maximum(cnt, 1.0),
                         jnp.ones((1, HALF), jnp.float32), _ONES_OUTER,
                         precision=lax.Precision.HIGHEST,
                         preferred_element_type=jnp.float32)
    o_ref[...] = jnp.concatenate([p_ref[0] * ic, p_ref[1] * ic], axis=1)


def kernel(x, edge_index, batch, emb, W1, b1, W2, b2):
    f32 = jnp.float32
    i32 = jnp.int32

    x0 = x[:, 0]
    xp = jnp.concatenate([x0, jnp.zeros((NP - N,), i32)])
    x2 = xp.reshape(NSG, 8, 128)

    src = edge_index[0]
    dst = edge_index[1]
    srcp = jnp.concatenate([src, jnp.zeros((EP - E,), i32)])
    dstp = jnp.concatenate([dst, jnp.full((EP - E,), N, i32)])
    ecomb = jnp.concatenate([srcp.reshape(ESG, 8, 128),
                             dstp.reshape(ESG, 8, 128)],
                            axis=1)

    batchp = jnp.concatenate([batch, jnp.full((NP - N,), G, i32)])
    batch2 = batchp.reshape(NSG, 8, 128)

    tab = emb.reshape(VOCAB, NC, HALF).transpose(1, 0, 2)

    deg_parts, cnt_parts, h0 = _prep(dstp, batchp, tab, x2)
    deg_parts = deg_parts.reshape(NW, NP)
    cnt_parts = cnt_parts.reshape(NW, GP)

    espec = pl.BlockSpec((NC, 1024, HALF), lambda i: (0, i, 0))
    dspec = pl.BlockSpec((1024, HALF), lambda i: (i, 0))
    eshape = jax.ShapeDtypeStruct((NC, NP, HALF), f32)

    g1, dinvb = _tc_call(
        _scale_body, (NB,),
        [pl.BlockSpec((NW, 1024), lambda i: (0, i)), espec],
        [espec, dspec],
        [eshape, jax.ShapeDtypeStruct((NP, HALF), f32)],
    )(deg_parts, h0)

    s1 = _spmm(g1, ecomb)

    g2 = _tc_call(
        _mlp_body, (NB,),
        [espec, dspec,
         pl.BlockSpec((NC, HALF, H1), lambda i: (0, 0, 0)),
         pl.BlockSpec((1, H1), lambda i: (0, 0)),
         pl.BlockSpec((H1, HALF), lambda i: (0, 0)),
         pl.BlockSpec((H1, HALF), lambda i: (0, 0))],
        espec, eshape,
    )(s1, dinvb, W1.reshape(NC, HALF, H1), b1.reshape(1, H1),
      W2[:, :HALF], W2[:, HALF:])

    pooled = _spmm_pool(g2, ecomb, batch2,
                        dinvb, b2, jnp.zeros((GP, HALF), f32))

    out = _tc_call(
        _mean_body, (1,),
        [pl.BlockSpec((NC, G, HALF), lambda i: (0, 0, 0)),
         pl.BlockSpec((NW, GP), lambda i: (0, 0))],
        pl.BlockSpec((G, H2), lambda i: (0, 0)),
        jax.ShapeDtypeStruct((G, H2), f32),
    )(pooled[:, :G, :], cnt_parts)

    return out
